# baseline (speedup 1.0000x reference)
import functools

import jax
import jax.numpy as jnp
from jax import lax
from jax.experimental import pallas as pl
from jax.experimental.pallas import tpu as pltpu
from jax.experimental.pallas import tpu_sc as plsc

B = 8192
D = 256
S = 10
H = 1024
E = 256

NW = 32
NS = 16
CH = 128
NSPLIT = 2
BSP = B // NSPLIT
NPW = BSP // NW
NODE_CH = NPW // CH
NCHUNKS = NODE_CH * S
BLK = 1024
NBLK = BSP // BLK


def _sc_body(nodes_hbm, nidx_hbm, table_hbm, nfeat_hbm, nrows_hbm,
             nidx_v, nodeidx_v, buf_v, sem, sem2):
    cid = lax.axis_index("c")
    sid = lax.axis_index("s")
    wid = sid * 2 + cid
    base = wid * NPW

    pltpu.sync_copy(nodes_hbm.at[wid], nodeidx_v)
    pltpu.sync_copy(nidx_hbm.at[wid], nidx_v)

    def gather(j, bb):
        if j < NODE_CH:
            return pltpu.async_copy(table_hbm.at[nodeidx_v.at[j]],
                                    buf_v.at[bb], sem if bb == 0 else sem2)
        return pltpu.async_copy(table_hbm.at[nidx_v.at[j - NODE_CH]],
                                buf_v.at[bb], sem if bb == 0 else sem2)

    def drain(j, bb):
        if j < NODE_CH:
            pltpu.sync_copy(buf_v.at[bb],
                            nfeat_hbm.at[pl.ds(base + j * CH, CH)])
        else:
            s, h = divmod(j - NODE_CH, NODE_CH)
            pltpu.sync_copy(buf_v.at[bb],
                            nrows_hbm.at[s].at[pl.ds(base + h * CH, CH)])

    total = NODE_CH + NCHUNKS
    cp = gather(0, 0)
    for j in range(total):
        cp.wait()
        if j + 1 < total:
            cp = gather(j + 1, (j + 1) % 2)
        drain(j, j % 2)


@jax.jit
def _sc_gather(nodes2d, nidx2d, table):
    mesh = plsc.VectorSubcoreMesh(core_axis_name="c", subcore_axis_name="s")
    f = pl.kernel(
        _sc_body,
        out_type=(
            jax.ShapeDtypeStruct((BSP, D), jnp.float32),
            jax.ShapeDtypeStruct((S, BSP, D), jnp.float32),
        ),
        mesh=mesh,
        scratch_types=[
            pltpu.VMEM((NCHUNKS, CH), jnp.int32),
            pltpu.VMEM((NODE_CH, CH), jnp.int32),
            pltpu.VMEM((2, CH, D), jnp.float32),
            pltpu.SemaphoreType.DMA,
            pltpu.SemaphoreType.DMA,
        ],
    )
    return f(nodes2d, nidx2d, table)


def _phase1_body(xn_ref, xr_ref, w2_ref, zt_ref, out_ref, mean_ref, acc_ref):
    i = pl.program_id(0)

    @pl.when(i == 0)
    def _init():
        acc_ref[0] = jnp.float32(0.0)
        acc_ref[1] = jnp.float32(0.0)

    zt = zt_ref[...]
    xm = xr_ref[0]
    for s in range(1, S):
        xm = xm + xr_ref[s]
    xm = xm * jnp.float32(1.0 / S)
    mean_ref[...] = xm
    tn = jnp.tanh(jnp.dot(xn_ref[...], w2_ref[...],
                          preferred_element_type=jnp.float32))
    tm = jnp.tanh(jnp.dot(xm, w2_ref[...],
                          preferred_element_type=jnp.float32))
    acc_ref[0] += jnp.sum(tn * zt)
    acc_ref[1] += jnp.sum(tm * zt)

    @pl.when(i == NBLK - 1)
    def _fin():
        out_ref[0] = acc_ref[0] / B
        out_ref[1] = acc_ref[1] / B


@jax.jit
def _phase1(nfeat, nrows, w2, zt):
    return pl.pallas_call(
        _phase1_body,
        grid=(NBLK,),
        in_specs=[
            pl.BlockSpec((BLK, D), lambda i: (i, 0)),
            pl.BlockSpec((S, BLK, D), lambda i: (0, i, 0)),
            pl.BlockSpec((D, H), lambda i: (0, 0)),
            pl.BlockSpec((1, H), lambda i: (0, 0)),
        ],
        out_specs=[
            pl.BlockSpec(memory_space=pltpu.SMEM),
            pl.BlockSpec((BLK, D), lambda i: (i, 0)),
        ],
        out_shape=[
            jax.ShapeDtypeStruct((2,), jnp.float32),
            jax.ShapeDtypeStruct((BSP, D), jnp.float32),
        ],
        scratch_shapes=[pltpu.SMEM((2,), jnp.float32)],
    )(nfeat, nrows, w2, zt)


def _phase2_body(s_ref, xn_ref, xm_ref, w_ref, out_ref):
    u0 = jnp.float32(0.0)
    u1 = jnp.float32(0.0)
    for k in range(NSPLIT):
        u0 += s_ref[k, 0]
        u1 += s_ref[k, 1]
    m = jnp.maximum(u0, u1)
    e0 = jnp.exp(u0 - m)
    e1 = jnp.exp(u1 - m)
    a0 = e0 / (e0 + e1)
    a1 = e1 / (e0 + e1)
    comb = jnp.maximum(a0 * xn_ref[...] + a1 * xm_ref[...], 0.0)
    out_ref[...] = jnp.maximum(
        lax.dot_general(w_ref[...], comb, (((1,), (1,)), ((), ())),
                        preferred_element_type=jnp.float32),
        0.0)


@jax.jit
def _phase2(scal, nfeat, nmean, w):
    return pl.pallas_call(
        _phase2_body,
        grid=(NBLK,),
        in_specs=[
            pl.BlockSpec(memory_space=pltpu.SMEM),
            pl.BlockSpec((BLK, D), lambda i: (i, 0)),
            pl.BlockSpec((BLK, D), lambda i: (i, 0)),
            pl.BlockSpec((E, D), lambda i: (0, 0)),
        ],
        out_specs=pl.BlockSpec((E, BLK), lambda i: (0, i)),
        out_shape=jax.ShapeDtypeStruct((E, BSP), jnp.float32),
    )(scal, nfeat, nmean, w)


def kernel(nodes, neigh_idx, features_table, weight, weight_2, z):
    nodes = nodes.astype(jnp.int32)
    neigh = neigh_idx.astype(jnp.int32)
    zt = z.reshape(1, H)
    feats, means, scals = [], [], []
    for k in range(NSPLIT):
        lo = k * BSP
        nodes2d = lax.dynamic_slice_in_dim(nodes, lo, BSP).reshape(
            NW, NODE_CH, CH)
        nidx2d = (lax.dynamic_slice_in_dim(neigh, lo, BSP)
                  .reshape(NW, NODE_CH, CH, S)
                  .transpose(0, 3, 1, 2)
                  .reshape(NW, NCHUNKS, CH))
        nfeat, nrows = _sc_gather(nodes2d, nidx2d, features_table)
        scal, nmean = _phase1(nfeat, nrows, weight_2, zt)
        feats.append(nfeat)
        means.append(nmean)
        scals.append(scal)
    scal = jnp.stack(scals)
    outs = [_phase2(scal, feats[k], means[k], weight) for k in range(NSPLIT)]
    return jnp.concatenate(outs, axis=1)


# speedup vs baseline: 1.0783x; 1.0783x over previous
import jax
import jax.numpy as jnp
from jax import lax
from jax.experimental import pallas as pl
from jax.experimental.pallas import tpu as pltpu
from jax.experimental.pallas import tpu_sc as plsc

B = 8192
D = 256
S = 10
H = 1024
E = 256

NW = 32
CH = 128
NPW = B // NW
NODE_CH = NPW // CH
NCHUNKS = NODE_CH * S
BLK = 1024
NBLK = B // BLK


def _sc_body(nodes_hbm, nidx_hbm, table_hbm, nfeat_hbm, nrows_hbm,
             nidx_v, nodeidx_v, buf_v, sem, sem2):
    cid = lax.axis_index("c")
    sid = lax.axis_index("s")
    wid = sid * 2 + cid
    base = wid * NPW

    pltpu.sync_copy(nodes_hbm.at[wid], nodeidx_v)
    pltpu.sync_copy(nidx_hbm.at[wid], nidx_v)

    def gather(j, bb):
        if j < NODE_CH:
            return pltpu.async_copy(table_hbm.at[nodeidx_v.at[j]],
                                    buf_v.at[bb], sem if bb == 0 else sem2)
        return pltpu.async_copy(table_hbm.at[nidx_v.at[j - NODE_CH]],
                                buf_v.at[bb], sem if bb == 0 else sem2)

    def drain(j, bb):
        if j < NODE_CH:
            pltpu.sync_copy(buf_v.at[bb],
                            nfeat_hbm.at[pl.ds(base + j * CH, CH)])
        else:
            s, h = divmod(j - NODE_CH, NODE_CH)
            pltpu.sync_copy(buf_v.at[bb],
                            nrows_hbm.at[s].at[pl.ds(base + h * CH, CH)])

    total = NODE_CH + NCHUNKS
    cp = gather(0, 0)
    for j in range(total):
        cp.wait()
        if j + 1 < total:
            cp = gather(j + 1, (j + 1) % 2)
        drain(j, j % 2)


@jax.jit
def _sc_gather(nodes2d, nidx2d, table):
    mesh = plsc.VectorSubcoreMesh(core_axis_name="c", subcore_axis_name="s")
    f = pl.kernel(
        _sc_body,
        out_type=(
            jax.ShapeDtypeStruct((B, D), jnp.float32),
            jax.ShapeDtypeStruct((S, B, D), jnp.float32),
        ),
        mesh=mesh,
        scratch_types=[
            pltpu.VMEM((NCHUNKS, CH), jnp.int32),
            pltpu.VMEM((NODE_CH, CH), jnp.int32),
            pltpu.VMEM((2, CH, D), jnp.float32),
            pltpu.SemaphoreType.DMA,
            pltpu.SemaphoreType.DMA,
        ],
    )
    return f(nodes2d, nidx2d, table)


def _tc_body(nfeat_ref, nrows_ref, w2_ref, zt_ref, w_ref, out_ref,
             mean_ref, acc_ref):
    p = pl.program_id(0)
    i = pl.program_id(1)
    off = pl.multiple_of(i * BLK, BLK)

    @pl.when((p == 0) & (i == 0))
    def _init():
        acc_ref[0] = jnp.float32(0.0)
        acc_ref[1] = jnp.float32(0.0)

    @pl.when(p == 0)
    def _reduce_and_sum():
        xm = nrows_ref[0]
        for s in range(1, S):
            xm = xm + nrows_ref[s]
        xm = xm * jnp.float32(1.0 / S)
        mean_ref[pl.ds(off, BLK), :] = xm
        zt = zt_ref[...]
        tn = jnp.tanh(jnp.dot(nfeat_ref[...], w2_ref[...],
                              preferred_element_type=jnp.float32))
        tm = jnp.tanh(jnp.dot(xm, w2_ref[...],
                              preferred_element_type=jnp.float32))
        acc_ref[0] += jnp.sum(tn * zt)
        acc_ref[1] += jnp.sum(tm * zt)

    @pl.when(p == 1)
    def _combine():
        u0 = acc_ref[0] / B
        u1 = acc_ref[1] / B
        m = jnp.maximum(u0, u1)
        e0 = jnp.exp(u0 - m)
        e1 = jnp.exp(u1 - m)
        a0 = e0 / (e0 + e1)
        a1 = e1 / (e0 + e1)
        comb = jnp.maximum(
            a0 * nfeat_ref[...] + a1 * mean_ref[pl.ds(off, BLK), :], 0.0)
        out_ref[...] = jnp.maximum(
            lax.dot_general(w_ref[...], comb, (((1,), (1,)), ((), ())),
                            preferred_element_type=jnp.float32),
            0.0)


@jax.jit
def _tc_fused(nfeat, nrows, w2, zt, w):
    return pl.pallas_call(
        _tc_body,
        grid=(2, NBLK),
        in_specs=[
            pl.BlockSpec((BLK, D), lambda p, i: (i, 0)),
            pl.BlockSpec((S, BLK, D),
                         lambda p, i: (0, jnp.where(p == 0, i, NBLK - 1), 0)),
            pl.BlockSpec((D, H), lambda p, i: (0, 0)),
            pl.BlockSpec((1, H), lambda p, i: (0, 0)),
            pl.BlockSpec((E, D), lambda p, i: (0, 0)),
        ],
        out_specs=pl.BlockSpec((E, BLK), lambda p, i: (0, i)),
        out_shape=jax.ShapeDtypeStruct((E, B), jnp.float32),
        scratch_shapes=[
            pltpu.VMEM((B, D), jnp.float32),
            pltpu.SMEM((2,), jnp.float32),
        ],
    )(nfeat, nrows, w2, zt, w)


def kernel(nodes, neigh_idx, features_table, weight, weight_2, z):
    nodes2d = nodes.astype(jnp.int32).reshape(NW, NODE_CH, CH)
    nidx2d = (neigh_idx.astype(jnp.int32)
              .reshape(NW, NODE_CH, CH, S)
              .transpose(0, 3, 1, 2)
              .reshape(NW, NCHUNKS, CH))
    nfeat, nrows = _sc_gather(nodes2d, nidx2d, features_table)
    return _tc_fused(nfeat, nrows, weight_2, z.reshape(1, H), weight)


# speedup vs baseline: 1.3160x; 1.2205x over previous
import jax
import jax.numpy as jnp
from jax import lax
from jax.experimental import pallas as pl
from jax.experimental.pallas import tpu as pltpu
from jax.experimental.pallas import tpu_sc as plsc

B = 8192
D = 256
S = 10
H = 1024
E = 256

NW = 32
NPW = B // NW
NB = 32
ROUNDS = NPW // NB
NH = S // 2
UNITS = ROUNDS * 2
NODE_CH = 64
NODE_N = NPW // NODE_CH
BLK = 1024
NBLK = B // BLK


def _sc_body(nodes_hbm, nidx_hbm, table_hbm, nfeat_hbm, nmean_hbm,
             nidx_v, nodeidx_v, bufs_v, accs_v, nodebuf_v,
             semA, semB, semN, semN2, semM):
    cid = lax.axis_index("c")
    sid = lax.axis_index("s")
    wid = sid * 2 + cid
    base = wid * NPW

    pltpu.sync_copy(nodes_hbm.at[wid], nodeidx_v)
    pltpu.sync_copy(nidx_hbm.at[wid], nidx_v)

    def fire_unit(u):
        p = u % 2
        sm = semA if p == 0 else semB
        return [pltpu.async_copy(table_hbm.at[nidx_v.at[u * NH + so]],
                                 bufs_v.at[p, so], sm)
                for so in range(NH)]

    ncp = pltpu.async_copy(table_hbm.at[nodeidx_v.at[0]], nodebuf_v, semN)
    node_w = []
    cps = fire_unit(0)
    acc_w = {}
    for u in range(UNITS):
        r, q = divmod(u, 2)
        accp = r % 2
        if q == 0 and r >= 2:
            acc_w.pop(accp).wait()
        for c in cps:
            c.wait()
        if u + 1 < UNITS:
            cps = fire_unit(u + 1)

        def nbody(n, carry):
            for d in range(D // 16):
                sl = pl.ds(d * 16, 16)
                a = bufs_v[q, 0, n, sl]
                for so in range(1, NH):
                    a = a + bufs_v[q, so, n, sl]
                if q == 0:
                    accs_v[accp, n, sl] = a
                else:
                    accs_v[accp, n, sl] = (
                        (accs_v[accp, n, sl] + a) * jnp.float32(1.0 / S))
            return carry

        lax.fori_loop(0, NB, nbody, 0)

        if q == 1:
            acc_w[accp] = pltpu.async_copy(
                accs_v.at[accp], nmean_hbm.at[pl.ds(base + r * NB, NB)],
                semM)

        if u % 2 == 1 and u // 2 < NODE_N:
            k = u // 2
            ncp.wait()
            node_w.append(pltpu.async_copy(
                nodebuf_v, nfeat_hbm.at[pl.ds(base + k * NODE_CH, NODE_CH)],
                semN2))
            if k + 1 < NODE_N:
                node_w[-1].wait()
                ncp = pltpu.async_copy(table_hbm.at[nodeidx_v.at[k + 1]],
                                       nodebuf_v, semN)

    for c in acc_w.values():
        c.wait()
    node_w[-1].wait()


@jax.jit
def _sc_gather(nodes2d, nidx2d, table):
    mesh = plsc.VectorSubcoreMesh(core_axis_name="c", subcore_axis_name="s")
    f = pl.kernel(
        _sc_body,
        out_type=(
            jax.ShapeDtypeStruct((B, D), jnp.float32),
            jax.ShapeDtypeStruct((B, D), jnp.float32),
        ),
        mesh=mesh,
        scratch_types=[
            pltpu.VMEM((UNITS * NH, NB), jnp.int32),
            pltpu.VMEM((NODE_N, NODE_CH), jnp.int32),
            pltpu.VMEM((2, NH, NB, D), jnp.float32),
            pltpu.VMEM((2, NB, D), jnp.float32),
            pltpu.VMEM((NODE_CH, D), jnp.float32),
            pltpu.SemaphoreType.DMA,
            pltpu.SemaphoreType.DMA,
            pltpu.SemaphoreType.DMA,
            pltpu.SemaphoreType.DMA,
            pltpu.SemaphoreType.DMA,
        ],
    )
    return f(nodes2d, nidx2d, table)


def _tc_body(nfeat_ref, nmean_ref, w2_ref, zt_ref, w_ref, out_ref, acc_ref):
    p = pl.program_id(0)

    @pl.when((p == 0) & (pl.program_id(1) == 0))
    def _init():
        acc_ref[0] = jnp.float32(0.0)
        acc_ref[1] = jnp.float32(0.0)

    @pl.when(p == 0)
    def _sums():
        zt = zt_ref[...]
        tn = jnp.tanh(jnp.dot(nfeat_ref[...], w2_ref[...],
                              preferred_element_type=jnp.float32))
        tm = jnp.tanh(jnp.dot(nmean_ref[...], w2_ref[...],
                              preferred_element_type=jnp.float32))
        acc_ref[0] += jnp.sum(tn * zt)
        acc_ref[1] += jnp.sum(tm * zt)

    @pl.when(p == 1)
    def _combine():
        u0 = acc_ref[0] / B
        u1 = acc_ref[1] / B
        m = jnp.maximum(u0, u1)
        e0 = jnp.exp(u0 - m)
        e1 = jnp.exp(u1 - m)
        a0 = e0 / (e0 + e1)
        a1 = e1 / (e0 + e1)
        comb = jnp.maximum(a0 * nfeat_ref[...] + a1 * nmean_ref[...], 0.0)
        out_ref[...] = jnp.maximum(
            lax.dot_general(w_ref[...], comb, (((1,), (1,)), ((), ())),
                            preferred_element_type=jnp.float32),
            0.0)


@jax.jit
def _tc_fused(nfeat, nmean, w2, zt, w):
    return pl.pallas_call(
        _tc_body,
        grid=(2, NBLK),
        in_specs=[
            pl.BlockSpec((BLK, D), lambda p, i: (i, 0)),
            pl.BlockSpec((BLK, D), lambda p, i: (i, 0)),
            pl.BlockSpec((D, H), lambda p, i: (0, 0)),
            pl.BlockSpec((1, H), lambda p, i: (0, 0)),
            pl.BlockSpec((E, D), lambda p, i: (0, 0)),
        ],
        out_specs=pl.BlockSpec((E, BLK), lambda p, i: (0, jnp.where(p == 0, 0, i))),
        out_shape=jax.ShapeDtypeStruct((E, B), jnp.float32),
        scratch_shapes=[
            pltpu.SMEM((2,), jnp.float32),
        ],
    )(nfeat, nmean, w2, zt, w)


def kernel(nodes, neigh_idx, features_table, weight, weight_2, z):
    nodes2d = nodes.astype(jnp.int32).reshape(NW, NODE_N, NODE_CH)
    nidx2d = (neigh_idx.astype(jnp.int32)
              .reshape(NW, ROUNDS, NB, 2, NH)
              .transpose(0, 1, 3, 4, 2)
              .reshape(NW, UNITS * NH, NB))
    nfeat, nmean = _sc_gather(nodes2d, nidx2d, features_table)
    return _tc_fused(nfeat, nmean, weight_2, z.reshape(1, H), weight)
